# Initial kernel scaffold; baseline (speedup 1.0000x reference)
import jax
import jax.numpy as jnp
from jax.experimental import pallas as pl


def kernel(embed, onehot_label):
    raise NotImplementedError("write your pallas kernel here")


# speedup vs baseline: 1.1134x; 1.1134x over previous
import functools

import jax
import jax.numpy as jnp
import numpy as np
from jax import lax
from jax.experimental import pallas as pl
from jax.experimental.pallas import tpu as pltpu
from jax.experimental.pallas import tpu_sc as plsc

_ROT0 = (13, 15, 26, 6)
_ROT1 = (17, 29, 16, 24)


def _tf2x32(k1, k2, c1, c2):
    k1 = np.asarray(k1, np.uint32)
    k2 = np.asarray(k2, np.uint32)
    x0 = np.asarray(c1, np.uint32)
    x1 = np.asarray(c2, np.uint32)
    ks2 = k1 ^ k2 ^ np.uint32(0x1BD11BDA)

    def rnds(x0, x1, rots):
        for r in rots:
            x0 = (x0 + x1).astype(np.uint32)
            x1 = ((x1 << np.uint32(r)) | (x1 >> np.uint32(32 - r))).astype(np.uint32)
            x1 = x0 ^ x1
        return x0, x1

    x0 = (x0 + k1).astype(np.uint32)
    x1 = (x1 + k2).astype(np.uint32)
    x0, x1 = rnds(x0, x1, _ROT0)
    x0 = (x0 + k2).astype(np.uint32)
    x1 = (x1 + ks2 + np.uint32(1)).astype(np.uint32)
    x0, x1 = rnds(x0, x1, _ROT1)
    x0 = (x0 + ks2).astype(np.uint32)
    x1 = (x1 + k1 + np.uint32(2)).astype(np.uint32)
    x0, x1 = rnds(x0, x1, _ROT0)
    x0 = (x0 + k1).astype(np.uint32)
    x1 = (x1 + k2 + np.uint32(3)).astype(np.uint32)
    x0, x1 = rnds(x0, x1, _ROT1)
    x0 = (x0 + k2).astype(np.uint32)
    x1 = (x1 + ks2 + np.uint32(4)).astype(np.uint32)
    x0, x1 = rnds(x0, x1, _ROT0)
    x0 = (x0 + ks2).astype(np.uint32)
    x1 = (x1 + k1 + np.uint32(5)).astype(np.uint32)
    return x0, x1


def _np_fold_in(key, d):
    a, b = _tf2x32(key[0], key[1], np.zeros(1, np.uint32),
                   np.full(1, d, np.uint32))
    return a[0], b[0]


def _np_random_bits(key, n):
    b1, b2 = _tf2x32(key[0], key[1], np.zeros(n, np.uint32),
                     np.arange(n, dtype=np.uint32))
    return b1 ^ b2


def _np_uniform(key, n):
    bits = _np_random_bits(key, n)
    fb = ((bits >> np.uint32(9)) | np.uint32(0x3F800000)).astype(np.uint32)
    return fb.view(np.float32) - np.float32(1.0)


def _np_permutation(key, n):
    x = np.arange(n)
    for _ in range(2):
        b1, b2 = _tf2x32(key[0], key[1], np.zeros(2, np.uint32),
                         np.arange(2, dtype=np.uint32))
        key, sub = (b1[0], b2[0]), (b1[1], b2[1])
        x = x[np.argsort(_np_random_bits(sub, n), kind="stable")]
    return x


B = 16384
D = 512
SUBW = 128
NSUB = 4
LW = 1000
NB = 5
NW = 32
KE = 128
KL = 64
ECH = (B * NSUB) // (NW * KE)


@functools.lru_cache(maxsize=None)
def _routing():
    key = (np.uint32(0), np.uint32(42))
    dec = _np_uniform(_np_fold_in(key, 0), B) < 0.5
    ps = [_np_permutation(_np_fold_in(key, i), B) for i in range(1, NSUB)]
    r = np.arange(B)
    src = np.stack([r] + [np.where(dec, p, r) for p in ps], axis=1)
    ge = (NSUB * src + np.arange(NSUB)[None, :]).reshape(-1)

    r0 = r[~dec]
    r1 = r[dec]
    copy_out, copy_src = NB * r0, r0
    quar_out = np.concatenate([NB * r1 + j for j in range(1, NB)])
    quar_src = np.concatenate([r1] + [p[r1] for p in ps])
    zero_out = np.concatenate(
        [NB * r1, (NB * r0[:, None] + np.arange(1, NB)[None, :]).reshape(-1)])

    def pad(a, k):
        m = NW * k
        n = -len(a) % m
        return (np.concatenate([a, np.repeat(a[-1:], n)])
                .astype(np.int32).reshape(NW, -1, k))

    return (pad(ge, KE), pad(copy_src, KL), pad(copy_out, KL),
            pad(quar_src, KL), pad(quar_out, KL), pad(zero_out, KL))


@functools.lru_cache(maxsize=None)
def _build(ncc, nqc, nzc):
    mesh = plsc.VectorSubcoreMesh(core_axis_name="c", subcore_axis_name="s")

    def body(embed4, label, ge, cs, co, qs, qo, zo, out_e, out_l,
             ge_v, cs_v, co_v, qs_v, qo_v, zo_v, ebuf, lbuf, sem):
        wid = lax.axis_index("s") * 2 + lax.axis_index("c")

        pltpu.sync_copy(ge.at[wid], ge_v)
        pltpu.sync_copy(cs.at[wid], cs_v)
        pltpu.sync_copy(co.at[wid], co_v)
        pltpu.sync_copy(qs.at[wid], qs_v)
        pltpu.sync_copy(qo.at[wid], qo_v)
        pltpu.sync_copy(zo.at[wid], zo_v)

        ebase = wid * (ECH * KE)

        def e_step(c, carry):
            pltpu.async_copy(embed4.at[ge_v.at[c]], ebuf, sem).wait()
            pltpu.sync_copy(ebuf, out_e.at[pl.ds(ebase + c * KE, KE)])
            return carry
        lax.fori_loop(0, ECH, e_step, 0)

        z16 = jnp.zeros((16,), jnp.float32)

        def zrow(i, carry):
            for j in range(LW // 16):
                lbuf[i, pl.ds(j * 16, 16)] = z16
            lbuf[i, pl.ds(LW - 16, 16)] = z16
            return carry
        lax.fori_loop(0, KL, zrow, 0)

        def z_step(c, carry):
            pltpu.async_copy(lbuf, out_l.at[zo_v.at[c]], sem).wait()
            return carry
        lax.fori_loop(0, nzc, z_step, 0)

        def c_step(c, carry):
            pltpu.async_copy(label.at[cs_v.at[c]], lbuf, sem).wait()
            pltpu.async_copy(lbuf, out_l.at[co_v.at[c]], sem).wait()
            return carry
        lax.fori_loop(0, ncc, c_step, 0)

        tailsel = jnp.arange(16) >= 8

        def q_step(c, carry):
            pltpu.async_copy(label.at[qs_v.at[c]], lbuf, sem).wait()

            def srow(i, icarry):
                for j in range(LW // 16):
                    lbuf[i, pl.ds(j * 16, 16)] = lbuf[i, pl.ds(j * 16, 16)] * 0.25
                v = lbuf[i, pl.ds(LW - 16, 16)]
                lbuf[i, pl.ds(LW - 16, 16)] = jnp.where(tailsel, v * 0.25, v)
                return icarry
            lax.fori_loop(0, KL, srow, 0)
            pltpu.async_copy(lbuf, out_l.at[qo_v.at[c]], sem).wait()
            return carry
        lax.fori_loop(0, nqc, q_step, 0)

    return pl.kernel(
        body,
        compiler_params=pltpu.CompilerParams(use_tc_tiling_on_sc=False),
        out_type=(
            jax.ShapeDtypeStruct((B * NSUB, SUBW), jnp.float32),
            jax.ShapeDtypeStruct((B * NB, LW), jnp.float32),
        ),
        mesh=mesh,
        scratch_types=[
            pltpu.VMEM((ECH, KE), jnp.int32),
            pltpu.VMEM((ncc, KL), jnp.int32),
            pltpu.VMEM((ncc, KL), jnp.int32),
            pltpu.VMEM((nqc, KL), jnp.int32),
            pltpu.VMEM((nqc, KL), jnp.int32),
            pltpu.VMEM((nzc, KL), jnp.int32),
            pltpu.VMEM((KE, SUBW), jnp.float32),
            pltpu.VMEM((KL, LW), jnp.float32),
            pltpu.SemaphoreType.DMA,
        ],
    )


def kernel(embed, onehot_label):
    ge, cs, co, qs, qo, zo = _routing()
    kfn = _build(cs.shape[1], qs.shape[1], zo.shape[1])
    embed4 = embed.reshape(B * NSUB, SUBW)
    out_e, out_l = kfn(embed4, onehot_label, ge, cs, co, qs, qo, zo)
    return out_e.reshape(B, D), out_l.reshape(B, NB * LW)


# speedup vs baseline: 1.1662x; 1.0475x over previous
import functools

import jax
import jax.numpy as jnp
import numpy as np
from jax import lax
from jax.experimental import pallas as pl
from jax.experimental.pallas import tpu as pltpu
from jax.experimental.pallas import tpu_sc as plsc

_ROT0 = (13, 15, 26, 6)
_ROT1 = (17, 29, 16, 24)


def _tf2x32(k1, k2, c1, c2):
    k1 = np.asarray(k1, np.uint32)
    k2 = np.asarray(k2, np.uint32)
    x0 = np.asarray(c1, np.uint32)
    x1 = np.asarray(c2, np.uint32)
    ks2 = k1 ^ k2 ^ np.uint32(0x1BD11BDA)

    def rnds(x0, x1, rots):
        for r in rots:
            x0 = (x0 + x1).astype(np.uint32)
            x1 = ((x1 << np.uint32(r)) | (x1 >> np.uint32(32 - r))).astype(np.uint32)
            x1 = x0 ^ x1
        return x0, x1

    x0 = (x0 + k1).astype(np.uint32)
    x1 = (x1 + k2).astype(np.uint32)
    x0, x1 = rnds(x0, x1, _ROT0)
    x0 = (x0 + k2).astype(np.uint32)
    x1 = (x1 + ks2 + np.uint32(1)).astype(np.uint32)
    x0, x1 = rnds(x0, x1, _ROT1)
    x0 = (x0 + ks2).astype(np.uint32)
    x1 = (x1 + k1 + np.uint32(2)).astype(np.uint32)
    x0, x1 = rnds(x0, x1, _ROT0)
    x0 = (x0 + k1).astype(np.uint32)
    x1 = (x1 + k2 + np.uint32(3)).astype(np.uint32)
    x0, x1 = rnds(x0, x1, _ROT1)
    x0 = (x0 + k2).astype(np.uint32)
    x1 = (x1 + ks2 + np.uint32(4)).astype(np.uint32)
    x0, x1 = rnds(x0, x1, _ROT0)
    x0 = (x0 + ks2).astype(np.uint32)
    x1 = (x1 + k1 + np.uint32(5)).astype(np.uint32)
    return x0, x1


def _np_fold_in(key, d):
    a, b = _tf2x32(key[0], key[1], np.zeros(1, np.uint32),
                   np.full(1, d, np.uint32))
    return a[0], b[0]


def _np_random_bits(key, n):
    b1, b2 = _tf2x32(key[0], key[1], np.zeros(n, np.uint32),
                     np.arange(n, dtype=np.uint32))
    return b1 ^ b2


def _np_uniform(key, n):
    bits = _np_random_bits(key, n)
    fb = ((bits >> np.uint32(9)) | np.uint32(0x3F800000)).astype(np.uint32)
    return fb.view(np.float32) - np.float32(1.0)


def _np_permutation(key, n):
    x = np.arange(n)
    for _ in range(2):
        b1, b2 = _tf2x32(key[0], key[1], np.zeros(2, np.uint32),
                         np.arange(2, dtype=np.uint32))
        key, sub = (b1[0], b2[0]), (b1[1], b2[1])
        x = x[np.argsort(_np_random_bits(sub, n), kind="stable")]
    return x


B = 16384
D = 512
SUBW = 128
NSUB = 4
LW = 1000
NB = 5
NW = 32
KE = 64
ECH = B // (NW * KE)
KL = 32
RPT = B // NW


def _mesh():
    return plsc.VectorSubcoreMesh(core_axis_name="c", subcore_axis_name="s")


@functools.lru_cache(maxsize=None)
def _routing():
    key = (np.uint32(0), np.uint32(42))
    dec = _np_uniform(_np_fold_in(key, 0), B) < 0.5
    ps = [_np_permutation(_np_fold_in(key, i), B) for i in range(1, NSUB)]
    r = np.arange(B)

    esrc = (np.stack([r] + [np.where(dec, p, r) for p in ps])
            .astype(np.int32).reshape(NSUB, NW, ECH, KE)
            .transpose(1, 0, 2, 3).copy())

    r0s = r[~dec]
    r1s = r[dec]
    copy_out, copy_src = NB * r0s, r0s
    quar_out = np.concatenate([NB * r1s + j for j in range(1, NB)])
    quar_src = np.concatenate([r1s] + [p[r1s] for p in ps])
    zero_out = np.concatenate(
        [NB * r1s, (NB * r0s[:, None] + np.arange(1, NB)[None, :]).reshape(-1)])

    def pad(a):
        m = 2 * NW * KL
        n = -len(a) % m
        return (np.concatenate([a, np.repeat(a[-1:], n)])
                .astype(np.int32).reshape(NW, -1, KL))

    return (esrc, pad(copy_src), pad(copy_out),
            pad(quar_src), pad(quar_out), pad(zero_out))


@functools.lru_cache(maxsize=None)
def _build_embed():
    def body(embed, esrc, out_e, esrc_v, eb0, eb1, gsem, wsem):
        wid = lax.axis_index("s") * 2 + lax.axis_index("c")
        base = wid * RPT
        pltpu.sync_copy(esrc.at[wid], esrc_v)

        def gather(c, eb):
            return [pltpu.async_copy(
                embed.at[esrc_v.at[i, c], pl.ds(i * SUBW, SUBW)],
                eb.at[:, pl.ds(i * SUBW, SUBW)], gsem)
                for i in range(NSUB)]

        def pair(p, carry):
            c0 = 2 * p
            g0 = gather(c0, eb0)
            for d in g0:
                d.wait()
            w0 = pltpu.async_copy(eb0, out_e.at[pl.ds(base + c0 * KE, KE)], wsem)
            g1 = gather(c0 + 1, eb1)
            for d in g1:
                d.wait()
            w0.wait()
            w1 = pltpu.async_copy(eb1, out_e.at[pl.ds(base + (c0 + 1) * KE, KE)], wsem)
            w1.wait()
            return carry
        lax.fori_loop(0, ECH // 2, pair, 0)

    return pl.kernel(
        body,
        out_type=jax.ShapeDtypeStruct((B, D), jnp.float32),
        mesh=_mesh(),
        scratch_types=[
            pltpu.VMEM((NSUB, ECH, KE), jnp.int32),
            pltpu.VMEM((KE, D), jnp.float32),
            pltpu.VMEM((KE, D), jnp.float32),
            pltpu.SemaphoreType.DMA,
            pltpu.SemaphoreType.DMA,
        ],
    )


@functools.lru_cache(maxsize=None)
def _build_label(ncc, nqc, nzc):
    def body(label, cs, co, qs, qo, zo, out_l,
             cs_v, co_v, qs_v, qo_v, zo_v, zbuf, ba, bb, gsem, ssem, zsem):
        wid = lax.axis_index("s") * 2 + lax.axis_index("c")

        pltpu.sync_copy(cs.at[wid], cs_v)
        pltpu.sync_copy(co.at[wid], co_v)
        pltpu.sync_copy(qs.at[wid], qs_v)
        pltpu.sync_copy(qo.at[wid], qo_v)
        pltpu.sync_copy(zo.at[wid], zo_v)

        z16 = jnp.zeros((16,), jnp.float32)

        def zrow(i, carry):
            for t in range(LW // 16):
                zbuf[i, pl.ds(t * 16, 16)] = z16
            zbuf[i, pl.ds(LW - 16, 16)] = z16
            return carry
        lax.fori_loop(0, KL, zrow, 0)

        zdescs = [pltpu.async_copy(zbuf, out_l.at[zo_v.at[z]], zsem)
                  for z in range(nzc)]

        def cpair(p, carry):
            c0 = 2 * p
            pltpu.async_copy(label.at[cs_v.at[c0]], ba, gsem).wait()
            sa = pltpu.async_copy(ba, out_l.at[co_v.at[c0]], ssem)
            pltpu.async_copy(label.at[cs_v.at[c0 + 1]], bb, gsem).wait()
            sa.wait()
            pltpu.async_copy(bb, out_l.at[co_v.at[c0 + 1]], ssem).wait()
            return carry
        lax.fori_loop(0, ncc // 2, cpair, 0)

        tailsel = jnp.arange(16) >= 8

        def scale(buf):
            def srow(i, carry):
                for t in range(LW // 16):
                    buf[i, pl.ds(t * 16, 16)] = buf[i, pl.ds(t * 16, 16)] * 0.25
                v = buf[i, pl.ds(LW - 16, 16)]
                buf[i, pl.ds(LW - 16, 16)] = jnp.where(tailsel, v * 0.25, v)
                return carry
            lax.fori_loop(0, KL, srow, 0)

        def qpair(p, carry):
            c0 = 2 * p
            pltpu.async_copy(label.at[qs_v.at[c0]], ba, gsem).wait()
            scale(ba)
            sa = pltpu.async_copy(ba, out_l.at[qo_v.at[c0]], ssem)
            pltpu.async_copy(label.at[qs_v.at[c0 + 1]], bb, gsem).wait()
            scale(bb)
            sa.wait()
            pltpu.async_copy(bb, out_l.at[qo_v.at[c0 + 1]], ssem).wait()
            return carry
        lax.fori_loop(0, nqc // 2, qpair, 0)

        for d in zdescs:
            d.wait()

    return pl.kernel(
        body,
        compiler_params=pltpu.CompilerParams(use_tc_tiling_on_sc=False),
        out_type=jax.ShapeDtypeStruct((B * NB, LW), jnp.float32),
        mesh=_mesh(),
        scratch_types=[
            pltpu.VMEM((ncc, KL), jnp.int32),
            pltpu.VMEM((ncc, KL), jnp.int32),
            pltpu.VMEM((nqc, KL), jnp.int32),
            pltpu.VMEM((nqc, KL), jnp.int32),
            pltpu.VMEM((nzc, KL), jnp.int32),
            pltpu.VMEM((KL, LW), jnp.float32),
            pltpu.VMEM((KL, LW), jnp.float32),
            pltpu.VMEM((KL, LW), jnp.float32),
            pltpu.SemaphoreType.DMA,
            pltpu.SemaphoreType.DMA,
            pltpu.SemaphoreType.DMA,
        ],
    )


def kernel(embed, onehot_label):
    esrc, cs, co, qs, qo, zo = _routing()
    out_e = _build_embed()(embed, esrc)
    out_l = _build_label(cs.shape[1], qs.shape[1], zo.shape[1])(
        onehot_label, cs, co, qs, qo, zo)
    return out_e, out_l.reshape(B, NB * LW)
